# baseline (speedup 1.0000x reference)
import jax
import jax.numpy as jnp
from jax import lax
from jax.experimental import pallas as pl
from jax.experimental.pallas import tpu as pltpu
from jax.experimental.pallas import tpu_sc as plsc

N = 10000
E = 160000
IN_DIM = 128
OUT_DIM = 128
REL_DIM = 16
AUXW = 32

NC, NS = 2, 16
NW = NC * NS
EPW = E // NW
B = 125
K = EPW // B
NPAD = 10240
RPS = NPAD // NS
ZR = 128


def _proj_body(h_ref, wn_ref, na_ref, h2_ref, h2s_ref):
    h2 = lax.dot_general(h_ref[...], wn_ref[...],
                         (((1,), (1,)), ((), ())),
                         preferred_element_type=jnp.float32)
    h2_ref[...] = h2
    h2s_ref[...] = na_ref[...] * h2


def _project(h, W_node, na2d):
    blk = 400
    grid = N // blk
    return pl.pallas_call(
        _proj_body,
        grid=(grid,),
        in_specs=[
            pl.BlockSpec((blk, IN_DIM), lambda i: (i, 0)),
            pl.BlockSpec((OUT_DIM, IN_DIM), lambda i: (0, 0)),
            pl.BlockSpec((blk, 1), lambda i: (i, 0)),
        ],
        out_specs=[
            pl.BlockSpec((blk, OUT_DIM), lambda i: (i, 0)),
            pl.BlockSpec((blk, OUT_DIM), lambda i: (i, 0)),
        ],
        out_shape=[
            jax.ShapeDtypeStruct((N, OUT_DIM), jnp.float32),
            jax.ShapeDtypeStruct((N, OUT_DIM), jnp.float32),
        ],
    )(h, W_node, na2d)


def _aux_body(eat_ref, attrt_ref, aux_ref):
    eat = eat_ref[...]
    attrt = attrt_ref[...]
    blk = eat.shape[1]
    place = (jnp.eye(REL_DIM, 128, dtype=jnp.float32)
             if REL_DIM <= 128 else None)
    attr_p = lax.dot_general(attrt, place, (((0,), (0,)), ((), ())),
                             preferred_element_type=jnp.float32)
    ea_col = lax.dot_general(eat, jnp.ones((1, 1), jnp.float32),
                             (((0,), (0,)), ((), ())),
                             preferred_element_type=jnp.float32)
    col = lax.broadcasted_iota(jnp.int32, (1, 128), 1)
    e16 = (col == REL_DIM).astype(jnp.float32)
    e17 = (col == REL_DIM + 1).astype(jnp.float32)
    aux_ref[...] = ea_col * (attr_p + e16) + e17


def _aux_build(eat, attr_t):
    blk = 1280
    grid = E // blk
    return pl.pallas_call(
        _aux_body,
        grid=(grid,),
        in_specs=[
            pl.BlockSpec((1, blk), lambda i: (0, i)),
            pl.BlockSpec((REL_DIM, blk), lambda i: (0, i)),
        ],
        out_specs=pl.BlockSpec((blk, 128), lambda i: (i, 0)),
        out_shape=jax.ShapeDtypeStruct((E, 128), jnp.float32),
    )(eat, attr_t)


def _zero_vmem(buf, rows, width):
    def _zero_row(i, _):
        def _zero_lane(j, _):
            buf[i, pl.ds(j * 16, 16)] = jnp.zeros((16,), jnp.float32)
            return 0
        return lax.fori_loop(0, width // 16, _zero_lane, 0)
    lax.fori_loop(0, rows, _zero_row, 0)


def _sc_wide_body(h2s_hbm, src_hbm, dst_hbm, out1_hbm,
                  src_v, dst_v, row_v, zb1, acc1):
    c = lax.axis_index("c")
    s = lax.axis_index("s")
    wid = s * NC + c

    _zero_vmem(zb1, ZR, OUT_DIM)
    for r in range(RPS // ZR):
        pltpu.sync_copy(zb1, acc1.at[pl.ds(s * RPS + r * ZR, ZR)])
    plsc.subcore_barrier()

    pltpu.sync_copy(src_hbm.at[wid], src_v)
    pltpu.sync_copy(dst_hbm.at[wid], dst_v)

    def _chunk(j, _):
        pltpu.sync_copy(h2s_hbm.at[src_v.at[j]], row_v)
        pltpu.sync_copy(row_v, acc1.at[dst_v.at[j]], add=True)
        return 0

    lax.fori_loop(0, K, _chunk, 0)
    plsc.subcore_barrier()

    pltpu.sync_copy(acc1.at[pl.ds(s * RPS, RPS)],
                    out1_hbm.at[c, pl.ds(s * RPS, RPS)])


def _sc_aux_body(aux_hbm, dst_hbm, outa_hbm, dst_v, aux_v, zba, acca):
    c = lax.axis_index("c")
    s = lax.axis_index("s")
    wid = s * NC + c

    _zero_vmem(zba, ZR, AUXW)
    for r in range(RPS // ZR):
        pltpu.sync_copy(zba, acca.at[pl.ds(s * RPS + r * ZR, ZR)])
    plsc.subcore_barrier()

    pltpu.sync_copy(dst_hbm.at[wid], dst_v)

    def _chunk(j, _):
        base = wid * EPW + j * B
        pltpu.sync_copy(aux_hbm.at[pl.ds(base, B), pl.ds(0, AUXW)], aux_v)
        pltpu.sync_copy(aux_v, acca.at[dst_v.at[j]], add=True)
        return 0

    lax.fori_loop(0, K, _chunk, 0)
    plsc.subcore_barrier()

    pltpu.sync_copy(acca.at[pl.ds(s * RPS, RPS)],
                    outa_hbm.at[c, pl.ds(s * RPS, RPS)])


def _sc_scatter2(h2s, aux128, src3, dst3):
    mesh = plsc.VectorSubcoreMesh(core_axis_name="c", subcore_axis_name="s")
    f_wide = pl.kernel(
        _sc_wide_body,
        out_type=jax.ShapeDtypeStruct((NC, NPAD, OUT_DIM), jnp.float32),
        mesh=mesh,
        scratch_types=[
            pltpu.VMEM((K, B), jnp.int32),
            pltpu.VMEM((K, B), jnp.int32),
            pltpu.VMEM((B, OUT_DIM), jnp.float32),
            pltpu.VMEM((ZR, OUT_DIM), jnp.float32),
            pltpu.VMEM_SHARED((NPAD, OUT_DIM), jnp.float32),
        ],
    )
    f_aux = pl.kernel(
        _sc_aux_body,
        out_type=jax.ShapeDtypeStruct((NC, NPAD, AUXW), jnp.float32),
        mesh=mesh,
        compiler_params=pltpu.CompilerParams(use_tc_tiling_on_sc=False),
        scratch_types=[
            pltpu.VMEM((K, B), jnp.int32),
            pltpu.VMEM((B, AUXW), jnp.float32),
            pltpu.VMEM((ZR, AUXW), jnp.float32),
            pltpu.VMEM_SHARED((NPAD, AUXW), jnp.float32),
        ],
    )
    return f_wide(h2s, src3, dst3), f_aux(aux128, dst3)


def _comb_body(o1_ref, oa_ref, h2_ref, na_ref, wr_ref, br_ref, wa_ref,
               ba_ref, out_ref):
    g1 = o1_ref[0] + o1_ref[1]
    a = oa_ref[0] + oa_ref[1]
    g2a = a[:, :REL_DIM]
    s_att = a[:, REL_DIM:REL_DIM + 1]
    deg = a[:, REL_DIM + 1:REL_DIM + 2]
    msg2 = lax.dot_general(g2a, wr_ref[...], (((1,), (1,)), ((), ())),
                           preferred_element_type=jnp.float32)
    msg2 = msg2 + s_att * br_ref[...]
    wa = wa_ref[...]
    pre = lax.dot_general(g1, wa[:, :OUT_DIM],
                          (((1,), (1,)), ((), ())),
                          preferred_element_type=jnp.float32)
    pre += lax.dot_general(msg2, wa[:, OUT_DIM:2 * OUT_DIM],
                           (((1,), (1,)), ((), ())),
                           preferred_element_type=jnp.float32)
    h2 = h2_ref[...]
    pre += lax.dot_general(h2, wa[:, 2 * OUT_DIM:],
                           (((1,), (1,)), ((), ())),
                           preferred_element_type=jnp.float32)
    pre += ba_ref[...]
    new = na_ref[...] * jax.nn.relu(pre)
    out_ref[...] = jnp.where(deg > 0, new, h2)


def _combine(out1, outa, h2, na2d, W_rel, br2d, W_apply, ba2d):
    blk = 400
    grid = N // blk
    return pl.pallas_call(
        _comb_body,
        grid=(grid,),
        in_specs=[
            pl.BlockSpec((NC, blk, OUT_DIM), lambda i: (0, i, 0)),
            pl.BlockSpec((NC, blk, AUXW), lambda i: (0, i, 0)),
            pl.BlockSpec((blk, OUT_DIM), lambda i: (i, 0)),
            pl.BlockSpec((blk, 1), lambda i: (i, 0)),
            pl.BlockSpec((OUT_DIM, REL_DIM), lambda i: (0, 0)),
            pl.BlockSpec((1, OUT_DIM), lambda i: (0, 0)),
            pl.BlockSpec((OUT_DIM, 3 * OUT_DIM), lambda i: (0, 0)),
            pl.BlockSpec((1, OUT_DIM), lambda i: (0, 0)),
        ],
        out_specs=pl.BlockSpec((blk, OUT_DIM), lambda i: (i, 0)),
        out_shape=jax.ShapeDtypeStruct((N, OUT_DIM), jnp.float32),
    )(out1, outa, h2, na2d, W_rel, br2d, W_apply, ba2d)


def kernel(h, edge_index, edge_attr, node_att, edge_att, W_node, W_rel,
           b_rel, W_apply, b_apply):
    na2d = node_att.reshape(N, 1)
    h2, h2s = _project(h, W_node, na2d)
    aux128 = _aux_build(edge_att.reshape(1, E), edge_attr.T)
    src3 = edge_index[0].reshape(NW, K, B)
    dst3 = edge_index[1].reshape(NW, K, B)
    out1, outa = _sc_scatter2(h2s, aux128, src3, dst3)
    return _combine(out1, outa, h2, na2d, W_rel, b_rel.reshape(1, OUT_DIM),
                    W_apply, b_apply.reshape(1, OUT_DIM))


# speedup vs baseline: 1.0372x; 1.0372x over previous
import jax
import jax.numpy as jnp
from jax import lax
from jax.experimental import pallas as pl
from jax.experimental.pallas import tpu as pltpu
from jax.experimental.pallas import tpu_sc as plsc

N = 10000
E = 160000
IN_DIM = 128
OUT_DIM = 128
REL_DIM = 16
AUXW = 32

NC, NS = 2, 16
NW = NC * NS
EPW = E // NW
B = 125
K = EPW // B
NPAD = 10240
RPS = NPAD // NS
ZR = 128


def _proj_body(h_ref, wn_ref, na_ref, h2_ref, h2s_ref):
    h2 = lax.dot_general(h_ref[...], wn_ref[...],
                         (((1,), (1,)), ((), ())),
                         preferred_element_type=jnp.float32)
    h2_ref[...] = h2
    h2s_ref[...] = na_ref[...] * h2


def _project(h, W_node, na2d):
    blk = 1000
    grid = N // blk
    return pl.pallas_call(
        _proj_body,
        grid=(grid,),
        in_specs=[
            pl.BlockSpec((blk, IN_DIM), lambda i: (i, 0)),
            pl.BlockSpec((OUT_DIM, IN_DIM), lambda i: (0, 0)),
            pl.BlockSpec((blk, 1), lambda i: (i, 0)),
        ],
        out_specs=[
            pl.BlockSpec((blk, OUT_DIM), lambda i: (i, 0)),
            pl.BlockSpec((blk, OUT_DIM), lambda i: (i, 0)),
        ],
        out_shape=[
            jax.ShapeDtypeStruct((N, OUT_DIM), jnp.float32),
            jax.ShapeDtypeStruct((N, OUT_DIM), jnp.float32),
        ],
    )(h, W_node, na2d)


def _aux_body(eat_ref, attrt_ref, aux_ref):
    eat = eat_ref[...]
    attrt = attrt_ref[...]
    place = jnp.eye(REL_DIM, 128, dtype=jnp.float32)
    col = lax.broadcasted_iota(jnp.int32, (1, 128), 1)
    e16 = (col == REL_DIM).astype(jnp.float32)
    e17 = (col == REL_DIM + 1).astype(jnp.float32)
    attr_p = lax.dot_general(eat * attrt, place, (((0,), (0,)), ((), ())),
                             preferred_element_type=jnp.float32)
    ea_p = lax.dot_general(eat, e16, (((0,), (0,)), ((), ())),
                           preferred_element_type=jnp.float32)
    aux_ref[...] = attr_p + ea_p + e17


def _aux_build(eat, attr_t):
    blk = 1280
    grid = E // blk
    return pl.pallas_call(
        _aux_body,
        grid=(grid,),
        in_specs=[
            pl.BlockSpec((1, blk), lambda i: (0, i)),
            pl.BlockSpec((REL_DIM, blk), lambda i: (0, i)),
        ],
        out_specs=pl.BlockSpec((blk, 128), lambda i: (i, 0)),
        out_shape=jax.ShapeDtypeStruct((E, 128), jnp.float32),
    )(eat, attr_t)


def _zero_vmem(buf, rows, width):
    def _zero_row(i, _):
        def _zero_lane(j, _):
            buf[i, pl.ds(j * 16, 16)] = jnp.zeros((16,), jnp.float32)
            return 0
        return lax.fori_loop(0, width // 16, _zero_lane, 0)
    lax.fori_loop(0, rows, _zero_row, 0)


def _sc_wide_body(h2s_hbm, src_hbm, dst_hbm, out1_hbm,
                  src_v, dst_v, row_v, zb1, acc1):
    c = lax.axis_index("c")
    s = lax.axis_index("s")
    wid = s * NC + c

    _zero_vmem(zb1, ZR, OUT_DIM)
    for r in range(RPS // ZR):
        pltpu.sync_copy(zb1, acc1.at[pl.ds(s * RPS + r * ZR, ZR)])
    plsc.subcore_barrier()

    pltpu.sync_copy(src_hbm.at[wid], src_v)
    pltpu.sync_copy(dst_hbm.at[wid], dst_v)

    def _chunk(j, _):
        pltpu.sync_copy(h2s_hbm.at[src_v.at[j]], row_v)
        pltpu.sync_copy(row_v, acc1.at[dst_v.at[j]], add=True)
        return 0

    lax.fori_loop(0, K, _chunk, 0)
    plsc.subcore_barrier()

    pltpu.sync_copy(acc1.at[pl.ds(s * RPS, RPS)],
                    out1_hbm.at[c, pl.ds(s * RPS, RPS)])


def _sc_aux_body(aux_hbm, dst_hbm, outa_hbm, dst_v, aux_v, zba, acca):
    c = lax.axis_index("c")
    s = lax.axis_index("s")
    wid = s * NC + c

    _zero_vmem(zba, ZR, AUXW)
    for r in range(RPS // ZR):
        pltpu.sync_copy(zba, acca.at[pl.ds(s * RPS + r * ZR, ZR)])
    plsc.subcore_barrier()

    pltpu.sync_copy(dst_hbm.at[wid], dst_v)

    def _chunk(j, _):
        base = wid * EPW + j * B
        pltpu.sync_copy(aux_hbm.at[pl.ds(base, B), pl.ds(0, AUXW)], aux_v)
        pltpu.sync_copy(aux_v, acca.at[dst_v.at[j]], add=True)
        return 0

    lax.fori_loop(0, K, _chunk, 0)
    plsc.subcore_barrier()

    pltpu.sync_copy(acca.at[pl.ds(s * RPS, RPS)],
                    outa_hbm.at[c, pl.ds(s * RPS, RPS)])


def _sc_scatter2(h2s, aux128, src3, dst3):
    mesh = plsc.VectorSubcoreMesh(core_axis_name="c", subcore_axis_name="s")
    f_wide = pl.kernel(
        _sc_wide_body,
        out_type=jax.ShapeDtypeStruct((NC, NPAD, OUT_DIM), jnp.float32),
        mesh=mesh,
        scratch_types=[
            pltpu.VMEM((K, B), jnp.int32),
            pltpu.VMEM((K, B), jnp.int32),
            pltpu.VMEM((B, OUT_DIM), jnp.float32),
            pltpu.VMEM((ZR, OUT_DIM), jnp.float32),
            pltpu.VMEM_SHARED((NPAD, OUT_DIM), jnp.float32),
        ],
    )
    f_aux = pl.kernel(
        _sc_aux_body,
        out_type=jax.ShapeDtypeStruct((NC, NPAD, AUXW), jnp.float32),
        mesh=mesh,
        compiler_params=pltpu.CompilerParams(use_tc_tiling_on_sc=False),
        scratch_types=[
            pltpu.VMEM((K, B), jnp.int32),
            pltpu.VMEM((B, AUXW), jnp.float32),
            pltpu.VMEM((ZR, AUXW), jnp.float32),
            pltpu.VMEM_SHARED((NPAD, AUXW), jnp.float32),
        ],
    )
    return f_wide(h2s, src3, dst3), f_aux(aux128, dst3)


def _comb_body(o1_ref, oa_ref, h2_ref, na_ref, wr_ref, br_ref, wa_ref,
               ba_ref, out_ref):
    g1 = o1_ref[0] + o1_ref[1]
    a = oa_ref[0] + oa_ref[1]
    g2a = a[:, :REL_DIM]
    s_att = a[:, REL_DIM:REL_DIM + 1]
    deg = a[:, REL_DIM + 1:REL_DIM + 2]
    msg2 = lax.dot_general(g2a, wr_ref[...], (((1,), (1,)), ((), ())),
                           preferred_element_type=jnp.float32)
    msg2 = msg2 + s_att * br_ref[...]
    wa = wa_ref[...]
    pre = lax.dot_general(g1, wa[:, :OUT_DIM],
                          (((1,), (1,)), ((), ())),
                          preferred_element_type=jnp.float32)
    pre += lax.dot_general(msg2, wa[:, OUT_DIM:2 * OUT_DIM],
                           (((1,), (1,)), ((), ())),
                           preferred_element_type=jnp.float32)
    h2 = h2_ref[...]
    pre += lax.dot_general(h2, wa[:, 2 * OUT_DIM:],
                           (((1,), (1,)), ((), ())),
                           preferred_element_type=jnp.float32)
    pre += ba_ref[...]
    new = na_ref[...] * jax.nn.relu(pre)
    out_ref[...] = jnp.where(deg > 0, new, h2)


def _combine(out1, outa, h2, na2d, W_rel, br2d, W_apply, ba2d):
    blk = 1000
    grid = N // blk
    return pl.pallas_call(
        _comb_body,
        grid=(grid,),
        in_specs=[
            pl.BlockSpec((NC, blk, OUT_DIM), lambda i: (0, i, 0)),
            pl.BlockSpec((NC, blk, AUXW), lambda i: (0, i, 0)),
            pl.BlockSpec((blk, OUT_DIM), lambda i: (i, 0)),
            pl.BlockSpec((blk, 1), lambda i: (i, 0)),
            pl.BlockSpec((OUT_DIM, REL_DIM), lambda i: (0, 0)),
            pl.BlockSpec((1, OUT_DIM), lambda i: (0, 0)),
            pl.BlockSpec((OUT_DIM, 3 * OUT_DIM), lambda i: (0, 0)),
            pl.BlockSpec((1, OUT_DIM), lambda i: (0, 0)),
        ],
        out_specs=pl.BlockSpec((blk, OUT_DIM), lambda i: (i, 0)),
        out_shape=jax.ShapeDtypeStruct((N, OUT_DIM), jnp.float32),
    )(out1, outa, h2, na2d, W_rel, br2d, W_apply, ba2d)


def kernel(h, edge_index, edge_attr, node_att, edge_att, W_node, W_rel,
           b_rel, W_apply, b_apply):
    na2d = node_att.reshape(N, 1)
    h2, h2s = _project(h, W_node, na2d)
    aux128 = _aux_build(edge_att.reshape(1, E), edge_attr.T)
    src3 = edge_index[0].reshape(NW, K, B)
    dst3 = edge_index[1].reshape(NW, K, B)
    out1, outa = _sc_scatter2(h2s, aux128, src3, dst3)
    return _combine(out1, outa, h2, na2d, W_rel, b_rel.reshape(1, OUT_DIM),
                    W_apply, b_apply.reshape(1, OUT_DIM))


# speedup vs baseline: 1.0674x; 1.0292x over previous
import jax
import jax.numpy as jnp
from jax import lax
from jax.experimental import pallas as pl
from jax.experimental.pallas import tpu as pltpu
from jax.experimental.pallas import tpu_sc as plsc

N = 10000
E = 160000
IN_DIM = 128
OUT_DIM = 128
REL_DIM = 16
AUXW = 32

NC, NS = 2, 16
NW = NC * NS
EPW = E // NW
B = 125
K = EPW // B
NPAD = 10240
RPS = NPAD // NS
ZR = 128


def _proj_body(h_ref, wn_ref, na_ref, h2_ref, h2s_ref):
    h2 = lax.dot_general(h_ref[...], wn_ref[...],
                         (((1,), (1,)), ((), ())),
                         preferred_element_type=jnp.float32)
    h2_ref[...] = h2
    h2s_ref[...] = na_ref[...] * h2


def _project(h, W_node, na2d):
    blk = 1000
    grid = N // blk
    return pl.pallas_call(
        _proj_body,
        grid=(grid,),
        in_specs=[
            pl.BlockSpec((blk, IN_DIM), lambda i: (i, 0)),
            pl.BlockSpec((OUT_DIM, IN_DIM), lambda i: (0, 0)),
            pl.BlockSpec((blk, 1), lambda i: (i, 0)),
        ],
        out_specs=[
            pl.BlockSpec((blk, OUT_DIM), lambda i: (i, 0)),
            pl.BlockSpec((blk, OUT_DIM), lambda i: (i, 0)),
        ],
        out_shape=[
            jax.ShapeDtypeStruct((N, OUT_DIM), jnp.float32),
            jax.ShapeDtypeStruct((N, OUT_DIM), jnp.float32),
        ],
    )(h, W_node, na2d)


def _aux_body(eat_ref, attrt_ref, aux_ref):
    eat = eat_ref[...]
    attrt = attrt_ref[...]
    place = jnp.eye(REL_DIM, 128, dtype=jnp.float32)
    col = lax.broadcasted_iota(jnp.int32, (1, 128), 1)
    e16 = (col == REL_DIM).astype(jnp.float32)
    e17 = (col == REL_DIM + 1).astype(jnp.float32)
    lhs = jnp.concatenate([eat * attrt, eat], axis=0)
    rhs = jnp.concatenate([place, e16], axis=0)
    aux_ref[...] = lax.dot_general(lhs, rhs, (((0,), (0,)), ((), ())),
                                   preferred_element_type=jnp.float32) + e17


def _aux_build(eat, attr_t):
    blk = 1280
    grid = E // blk
    return pl.pallas_call(
        _aux_body,
        grid=(grid,),
        in_specs=[
            pl.BlockSpec((1, blk), lambda i: (0, i)),
            pl.BlockSpec((REL_DIM, blk), lambda i: (0, i)),
        ],
        out_specs=pl.BlockSpec((blk, 128), lambda i: (i, 0)),
        out_shape=jax.ShapeDtypeStruct((E, 128), jnp.float32),
        compiler_params=pltpu.CompilerParams(fuse_transposed_lhs_in_matmul=True),
    )(eat, attr_t)


def _zero_vmem(buf, rows, width):
    def _zero_row(i, _):
        def _zero_lane(j, _):
            buf[i, pl.ds(j * 16, 16)] = jnp.zeros((16,), jnp.float32)
            return 0
        return lax.fori_loop(0, width // 16, _zero_lane, 0)
    lax.fori_loop(0, rows, _zero_row, 0)


def _sc_wide_body(h2s_hbm, src_hbm, dst_hbm, out1_hbm,
                  src_v, dst_v, row_v, zb1, acc1):
    c = lax.axis_index("c")
    s = lax.axis_index("s")
    wid = s * NC + c

    _zero_vmem(zb1, ZR, OUT_DIM)
    for r in range(RPS // ZR):
        pltpu.sync_copy(zb1, acc1.at[pl.ds(s * RPS + r * ZR, ZR)])
    plsc.subcore_barrier()

    pltpu.sync_copy(src_hbm.at[wid], src_v)
    pltpu.sync_copy(dst_hbm.at[wid], dst_v)

    def _chunk(j, _):
        pltpu.sync_copy(h2s_hbm.at[src_v.at[j]], row_v)
        pltpu.sync_copy(row_v, acc1.at[dst_v.at[j]], add=True)
        return 0

    lax.fori_loop(0, K, _chunk, 0)
    plsc.subcore_barrier()

    pltpu.sync_copy(acc1.at[pl.ds(s * RPS, RPS)],
                    out1_hbm.at[c, pl.ds(s * RPS, RPS)])


def _sc_aux_body(aux_hbm, dst_hbm, outa_hbm, dst_v, aux_v, zba, acca):
    c = lax.axis_index("c")
    s = lax.axis_index("s")
    wid = s * NC + c

    _zero_vmem(zba, ZR, AUXW)
    for r in range(RPS // ZR):
        pltpu.sync_copy(zba, acca.at[pl.ds(s * RPS + r * ZR, ZR)])
    plsc.subcore_barrier()

    pltpu.sync_copy(dst_hbm.at[wid], dst_v)

    def _chunk(j, _):
        base = wid * EPW + j * B
        pltpu.sync_copy(aux_hbm.at[pl.ds(base, B), pl.ds(0, AUXW)], aux_v)
        pltpu.sync_copy(aux_v, acca.at[dst_v.at[j]], add=True)
        return 0

    lax.fori_loop(0, K, _chunk, 0)
    plsc.subcore_barrier()

    pltpu.sync_copy(acca.at[pl.ds(s * RPS, RPS)],
                    outa_hbm.at[c, pl.ds(s * RPS, RPS)])


def _sc_scatter2(h2s, aux128, src3, dst3):
    mesh = plsc.VectorSubcoreMesh(core_axis_name="c", subcore_axis_name="s")
    f_wide = pl.kernel(
        _sc_wide_body,
        out_type=jax.ShapeDtypeStruct((NC, NPAD, OUT_DIM), jnp.float32),
        mesh=mesh,
        scratch_types=[
            pltpu.VMEM((K, B), jnp.int32),
            pltpu.VMEM((K, B), jnp.int32),
            pltpu.VMEM((B, OUT_DIM), jnp.float32),
            pltpu.VMEM((ZR, OUT_DIM), jnp.float32),
            pltpu.VMEM_SHARED((NPAD, OUT_DIM), jnp.float32),
        ],
    )
    f_aux = pl.kernel(
        _sc_aux_body,
        out_type=jax.ShapeDtypeStruct((NC, NPAD, AUXW), jnp.float32),
        mesh=mesh,
        compiler_params=pltpu.CompilerParams(use_tc_tiling_on_sc=False),
        scratch_types=[
            pltpu.VMEM((K, B), jnp.int32),
            pltpu.VMEM((B, AUXW), jnp.float32),
            pltpu.VMEM((ZR, AUXW), jnp.float32),
            pltpu.VMEM_SHARED((NPAD, AUXW), jnp.float32),
        ],
    )
    return f_wide(h2s, src3, dst3), f_aux(aux128, dst3)


def _comb_body(o1_ref, oa_ref, h2_ref, na_ref, wr_ref, br_ref, wa_ref,
               ba_ref, out_ref):
    g1 = o1_ref[0] + o1_ref[1]
    a = oa_ref[0] + oa_ref[1]
    g2a = a[:, :REL_DIM]
    s_att = a[:, REL_DIM:REL_DIM + 1]
    deg = a[:, REL_DIM + 1:REL_DIM + 2]
    msg2 = lax.dot_general(g2a, wr_ref[...], (((1,), (1,)), ((), ())),
                           preferred_element_type=jnp.float32)
    msg2 = msg2 + s_att * br_ref[...]
    wa = wa_ref[...]
    pre = lax.dot_general(g1, wa[:, :OUT_DIM],
                          (((1,), (1,)), ((), ())),
                          preferred_element_type=jnp.float32)
    pre += lax.dot_general(msg2, wa[:, OUT_DIM:2 * OUT_DIM],
                           (((1,), (1,)), ((), ())),
                           preferred_element_type=jnp.float32)
    h2 = h2_ref[...]
    pre += lax.dot_general(h2, wa[:, 2 * OUT_DIM:],
                           (((1,), (1,)), ((), ())),
                           preferred_element_type=jnp.float32)
    pre += ba_ref[...]
    new = na_ref[...] * jax.nn.relu(pre)
    out_ref[...] = jnp.where(deg > 0, new, h2)


def _combine(out1, outa, h2, na2d, W_rel, br2d, W_apply, ba2d):
    blk = 1000
    grid = N // blk
    return pl.pallas_call(
        _comb_body,
        grid=(grid,),
        in_specs=[
            pl.BlockSpec((NC, blk, OUT_DIM), lambda i: (0, i, 0)),
            pl.BlockSpec((NC, blk, AUXW), lambda i: (0, i, 0)),
            pl.BlockSpec((blk, OUT_DIM), lambda i: (i, 0)),
            pl.BlockSpec((blk, 1), lambda i: (i, 0)),
            pl.BlockSpec((OUT_DIM, REL_DIM), lambda i: (0, 0)),
            pl.BlockSpec((1, OUT_DIM), lambda i: (0, 0)),
            pl.BlockSpec((OUT_DIM, 3 * OUT_DIM), lambda i: (0, 0)),
            pl.BlockSpec((1, OUT_DIM), lambda i: (0, 0)),
        ],
        out_specs=pl.BlockSpec((blk, OUT_DIM), lambda i: (i, 0)),
        out_shape=jax.ShapeDtypeStruct((N, OUT_DIM), jnp.float32),
    )(out1, outa, h2, na2d, W_rel, br2d, W_apply, ba2d)


def kernel(h, edge_index, edge_attr, node_att, edge_att, W_node, W_rel,
           b_rel, W_apply, b_apply):
    na2d = node_att.reshape(N, 1)
    h2, h2s = _project(h, W_node, na2d)
    aux128 = _aux_build(edge_att.reshape(1, E), edge_attr.T)
    src3 = edge_index[0].reshape(NW, K, B)
    dst3 = edge_index[1].reshape(NW, K, B)
    out1, outa = _sc_scatter2(h2s, aux128, src3, dst3)
    return _combine(out1, outa, h2, na2d, W_rel, b_rel.reshape(1, OUT_DIM),
                    W_apply, b_apply.reshape(1, OUT_DIM))
